# baseline (speedup 1.0000x reference)
import functools

import jax
import jax.numpy as jnp
from jax.experimental import pallas as pl
from jax.experimental.pallas import tpu as pltpu
from jax.experimental.pallas import tpu_sc as plsc

LANES = 16
BLOCK_ROWS = 16
ROW = 200
CHUNK_STARTS = tuple(range(0, ROW - LANES + 1, LANES)) + (ROW - LANES,)


def kernel(inputs, vals):
    rows, cols = inputs.shape
    x = inputs.astype(jnp.int32)

    vals_i32 = vals.astype(jnp.int32)
    table16 = jnp.full((LANES,), -1, dtype=jnp.int32)
    table16 = table16.at[: vals_i32.shape[0]].set(vals_i32)

    mesh = plsc.VectorSubcoreMesh(core_axis_name="c", subcore_axis_name="s")

    dnums = jax.lax.GatherDimensionNumbers(
        offset_dims=(),
        collapsed_slice_dims=(0,),
        start_index_map=(0,),
    )

    @functools.partial(
        pl.kernel,
        out_type=jax.ShapeDtypeStruct((rows, cols), jnp.int32),
        mesh=mesh,
        scratch_types=[
            pltpu.VMEM((LANES,), jnp.int32),
            pltpu.SemaphoreType.DMA,
        ],
    )
    def _lookup(table_hbm, x_hbm, o_hbm, table_vmem, sem):
        pltpu.async_copy(table_hbm, table_vmem, sem).wait()

        def body(x_vmem, o_vmem):
            tbl = table_vmem[...]

            def translate(chunk):
                idx = chunk & (LANES - 1)
                return jax.lax.gather(
                    tbl,
                    idx[:, None],
                    dnums,
                    slice_sizes=(1,),
                    mode=jax.lax.GatherScatterMode.PROMISE_IN_BOUNDS,
                )

            sites = [
                (r, c) for r in range(BLOCK_ROWS) for c in CHUNK_STARTS
            ]
            GROUP = 16
            for g in range(0, len(sites), GROUP):
                grp = sites[g : g + GROUP]
                loaded = [x_vmem[r, pl.ds(c, LANES)] for (r, c) in grp]
                results = [translate(chunk) for chunk in loaded]
                for (r, c), res in zip(grp, results):
                    o_vmem[r, pl.ds(c, LANES)] = res

        pltpu.emit_pipeline(
            body,
            grid=(rows // BLOCK_ROWS,),
            in_specs=[pl.BlockSpec((BLOCK_ROWS, ROW), lambda i: (i, 0))],
            out_specs=[pl.BlockSpec((BLOCK_ROWS, ROW), lambda i: (i, 0))],
            core_axis_name=("c", "s"),
            dimension_semantics=(pltpu.PARALLEL,),
        )(x_hbm, o_hbm)

    out = _lookup(table16, x)
    return out.astype(vals.dtype)


# speedup vs baseline: 1.0825x; 1.0825x over previous
import functools

import jax
import jax.numpy as jnp
from jax.experimental import pallas as pl
from jax.experimental.pallas import tpu as pltpu
from jax.experimental.pallas import tpu_sc as plsc

LANES = 16
BLOCK_ROWS = 32
ROW = 200
CHUNK_STARTS = tuple(range(0, ROW - LANES + 1, LANES)) + (ROW - LANES,)


def kernel(inputs, vals):
    rows, cols = inputs.shape
    x = inputs.astype(jnp.int32)

    vals_i32 = vals.astype(jnp.int32)
    table16 = jnp.full((LANES,), -1, dtype=jnp.int32)
    table16 = table16.at[: vals_i32.shape[0]].set(vals_i32)

    mesh = plsc.VectorSubcoreMesh(core_axis_name="c", subcore_axis_name="s")

    dnums = jax.lax.GatherDimensionNumbers(
        offset_dims=(),
        collapsed_slice_dims=(0,),
        start_index_map=(0,),
    )

    @functools.partial(
        pl.kernel,
        out_type=jax.ShapeDtypeStruct((rows, cols), jnp.int32),
        mesh=mesh,
        scratch_types=[
            pltpu.VMEM((LANES,), jnp.int32),
            pltpu.SemaphoreType.DMA,
        ],
    )
    def _lookup(table_hbm, x_hbm, o_hbm, table_vmem, sem):
        pltpu.async_copy(table_hbm, table_vmem, sem).wait()

        def body(x_vmem, o_vmem):
            tbl = table_vmem[...]

            def translate(chunk):
                idx = chunk & (LANES - 1)
                return jax.lax.gather(
                    tbl,
                    idx[:, None],
                    dnums,
                    slice_sizes=(1,),
                    mode=jax.lax.GatherScatterMode.PROMISE_IN_BOUNDS,
                )

            @pl.loop(0, BLOCK_ROWS, unroll=4)
            def _(r):
                loaded = [x_vmem[r, pl.ds(c, LANES)] for c in CHUNK_STARTS]
                results = [translate(chunk) for chunk in loaded]
                for c, res in zip(CHUNK_STARTS, results):
                    o_vmem[r, pl.ds(c, LANES)] = res

        pltpu.emit_pipeline(
            body,
            grid=(rows // BLOCK_ROWS,),
            in_specs=[pl.BlockSpec((BLOCK_ROWS, ROW), lambda i: (i, 0))],
            out_specs=[pl.BlockSpec((BLOCK_ROWS, ROW), lambda i: (i, 0))],
            core_axis_name=("c", "s"),
            dimension_semantics=(pltpu.PARALLEL,),
        )(x_hbm, o_hbm)

    out = _lookup(table16, x)
    return out.astype(vals.dtype)


# speedup vs baseline: 1.0837x; 1.0011x over previous
import functools

import jax
import jax.numpy as jnp
from jax import lax
from jax.experimental import pallas as pl
from jax.experimental.pallas import tpu as pltpu
from jax.experimental.pallas import tpu_sc as plsc

LANES = 16
NUM_CORES = 2
NUM_SUBCORES = 16
NUM_WORKERS = NUM_CORES * NUM_SUBCORES
STEP_ROWS = 32
ROW = 200
CHUNK_STARTS = tuple(range(0, ROW - LANES + 1, LANES)) + (ROW - LANES,)


def kernel(inputs, vals):
    rows, cols = inputs.shape
    x = inputs.astype(jnp.int32)

    rows_per_worker = rows // NUM_WORKERS
    nsteps = rows_per_worker // STEP_ROWS

    vals_i32 = vals.astype(jnp.int32)
    table16 = jnp.full((LANES,), -1, dtype=jnp.int32)
    table16 = table16.at[: vals_i32.shape[0]].set(vals_i32)

    mesh = plsc.VectorSubcoreMesh(core_axis_name="c", subcore_axis_name="s")

    dnums = jax.lax.GatherDimensionNumbers(
        offset_dims=(),
        collapsed_slice_dims=(0,),
        start_index_map=(0,),
    )

    buf = pltpu.VMEM((STEP_ROWS, ROW), jnp.int32)

    @functools.partial(
        pl.kernel,
        out_type=jax.ShapeDtypeStruct((rows, cols), jnp.int32),
        mesh=mesh,
        scratch_types=[
            pltpu.VMEM((LANES,), jnp.int32),
            buf,
            buf,
            buf,
            buf,
            pltpu.SemaphoreType.DMA,
            pltpu.SemaphoreType.DMA,
            pltpu.SemaphoreType.DMA,
            pltpu.SemaphoreType.DMA,
            pltpu.SemaphoreType.DMA,
        ],
    )
    def _lookup(
        table_hbm, x_hbm, o_hbm, table_vmem,
        xb0, xb1, ob0, ob1, sem_t, si0, si1, so0, so1,
    ):
        wid = lax.axis_index("s") * NUM_CORES + lax.axis_index("c")
        base = wid * rows_per_worker
        pltpu.async_copy(table_hbm, table_vmem, sem_t).wait()
        tbl = table_vmem[...]

        def in_rows(s):
            return x_hbm.at[pl.ds(base + s * STEP_ROWS, STEP_ROWS)]

        def out_rows(s):
            return o_hbm.at[pl.ds(base + s * STEP_ROWS, STEP_ROWS)]

        def translate(chunk):
            idx = chunk & (LANES - 1)
            return jax.lax.gather(
                tbl,
                idx[:, None],
                dnums,
                slice_sizes=(1,),
                mode=jax.lax.GatherScatterMode.PROMISE_IN_BOUNDS,
            )

        def compute(x_vmem, o_vmem):
            @pl.loop(0, STEP_ROWS, unroll=4)
            def _(r):
                loaded = [x_vmem[r, pl.ds(c, LANES)] for c in CHUNK_STARTS]
                results = [translate(chunk) for chunk in loaded]
                for c, res in zip(CHUNK_STARTS, results):
                    o_vmem[r, pl.ds(c, LANES)] = res

        pltpu.make_async_copy(in_rows(0), xb0, si0).start()
        pltpu.make_async_copy(in_rows(1), xb1, si1).start()

        @pl.loop(0, nsteps, step=2)
        def _(s):
            for parity, (xb, ob, si, so) in enumerate(
                ((xb0, ob0, si0, so0), (xb1, ob1, si1, so1))
            ):
                step = s + parity
                pltpu.make_async_copy(in_rows(step), xb, si).wait()

                @pl.when(step >= 2)
                def _():
                    pltpu.make_async_copy(ob, out_rows(step - 2), so).wait()

                compute(xb, ob)
                pltpu.make_async_copy(ob, out_rows(step), so).start()

                @pl.when(step + 2 < nsteps)
                def _():
                    pltpu.make_async_copy(in_rows(step + 2), xb, si).start()

        pltpu.make_async_copy(ob0, out_rows(nsteps - 2), so0).wait()
        pltpu.make_async_copy(ob1, out_rows(nsteps - 1), so1).wait()

    out = _lookup(table16, x)
    return out.astype(vals.dtype)
